# Initial kernel scaffold; baseline (speedup 1.0000x reference)
import jax
import jax.numpy as jnp
from jax.experimental import pallas as pl


def kernel(positions, edge_indices, shift):
    raise NotImplementedError("write your pallas kernel here")


# speedup vs baseline: 1.3256x; 1.3256x over previous
import jax
import jax.numpy as jnp
from jax import lax
from jax.experimental import pallas as pl
from jax.experimental.pallas import tpu as pltpu
from jax.experimental.pallas import tpu_sc as plsc

N_CORES = 2
N_SUBCORES = 16
LANES = 16
N_WORKERS = N_CORES * N_SUBCORES

CHUNK = 2000


def _edge_body(px, py, pz, row_hbm, col_hbm, shift_hbm, dist_hbm, vec_hbm,
               idx_r, idx_c, rx, ry, rz, cx, cy, cz, sh, od, ov, sem):
    n_edges = row_hbm.shape[0]
    per_worker = n_edges // N_WORKERS
    n_chunks = per_worker // CHUNK
    wid = lax.axis_index("s") * N_CORES + lax.axis_index("c")
    wbase = wid * per_worker

    def chunk_body(k, carry):
        base = wbase + k * CHUNK
        pltpu.sync_copy(row_hbm.at[pl.ds(base, CHUNK)], idx_r)
        pltpu.sync_copy(col_hbm.at[pl.ds(base, CHUNK)], idx_c)
        pltpu.sync_copy(shift_hbm.at[pl.ds(base * 3, CHUNK * 3)], sh)
        cps = [pltpu.async_copy(px.at[idx_r], rx, sem),
               pltpu.async_copy(py.at[idx_r], ry, sem),
               pltpu.async_copy(pz.at[idx_r], rz, sem),
               pltpu.async_copy(px.at[idx_c], cx, sem),
               pltpu.async_copy(py.at[idx_c], cy, sem),
               pltpu.async_copy(pz.at[idx_c], cz, sem)]
        for cp in cps:
            cp.wait()

        lane3 = lax.iota(jnp.int32, LANES) * 3

        def vec_body(j, c2):
            e16 = j * LANES
            s48 = e16 * 3
            m = idx_r[pl.ds(e16, LANES)] != idx_c[pl.ds(e16, LANES)]
            d = []
            for comp, (rv, cv) in enumerate(((rx, cx), (ry, cy), (rz, cz))):
                sid = s48 + comp + lane3
                sv = plsc.load_gather(sh, [sid])
                d.append(rv[pl.ds(e16, LANES)] - cv[pl.ds(e16, LANES)] - sv)
            sq = d[0] * d[0] + d[1] * d[1] + d[2] * d[2]
            bits = plsc.bitcast(sq, jnp.int32)
            y = plsc.bitcast(jnp.int32(0x5F3759DF) - (bits >> 1), jnp.float32)
            for _ in range(3):
                y = y * (1.5 - 0.5 * sq * y * y)
            od[pl.ds(e16, LANES)] = jnp.where(m, sq * y, 0.0)
            ym = jnp.where(m, y, 1.0)
            for comp in range(3):
                plsc.store_scatter(ov, [s48 + comp + lane3], d[comp] * ym)
            return c2

        lax.fori_loop(0, CHUNK // LANES, vec_body, 0)
        pltpu.sync_copy(od, dist_hbm.at[pl.ds(base, CHUNK)])
        pltpu.sync_copy(ov, vec_hbm.at[pl.ds(base * 3, CHUNK * 3)])
        return carry

    lax.fori_loop(0, n_chunks, chunk_body, 0)


def kernel(positions, edge_indices, shift):
    row = edge_indices[0].astype(jnp.int32)
    col = edge_indices[1].astype(jnp.int32)
    px = jnp.asarray(positions[:, 0])
    py = jnp.asarray(positions[:, 1])
    pz = jnp.asarray(positions[:, 2])
    n_edges = row.shape[0]
    mesh = plsc.VectorSubcoreMesh(core_axis_name="c", subcore_axis_name="s")
    fn = pl.kernel(
        _edge_body,
        out_type=(jax.ShapeDtypeStruct((n_edges,), jnp.float32),
                  jax.ShapeDtypeStruct((n_edges * 3,), jnp.float32)),
        mesh=mesh,
        compiler_params=pltpu.CompilerParams(needs_layout_passes=False),
        scratch_types=[
            pltpu.VMEM((CHUNK,), jnp.int32),
            pltpu.VMEM((CHUNK,), jnp.int32),
            pltpu.VMEM((CHUNK,), jnp.float32),
            pltpu.VMEM((CHUNK,), jnp.float32),
            pltpu.VMEM((CHUNK,), jnp.float32),
            pltpu.VMEM((CHUNK,), jnp.float32),
            pltpu.VMEM((CHUNK,), jnp.float32),
            pltpu.VMEM((CHUNK,), jnp.float32),
            pltpu.VMEM((CHUNK * 3,), jnp.float32),
            pltpu.VMEM((CHUNK,), jnp.float32),
            pltpu.VMEM((CHUNK * 3,), jnp.float32),
            pltpu.SemaphoreType.DMA,
        ],
    )
    dist, vec = fn(px, py, pz, row, col, shift.reshape(-1))
    return (edge_indices, dist, vec.reshape(n_edges, 3))


# speedup vs baseline: 1.3923x; 1.0503x over previous
import jax
import jax.numpy as jnp
from jax import lax
from jax.experimental import pallas as pl
from jax.experimental.pallas import tpu as pltpu
from jax.experimental.pallas import tpu_sc as plsc

N_CORES = 2
N_SUBCORES = 16
LANES = 16
N_WORKERS = N_CORES * N_SUBCORES

CHUNK = 2000


def _edge_body(px, py, pz, row_hbm, col_hbm, shift_hbm, dist_hbm, vec_hbm,
               idx_r, idx_c, rx, ry, rz, cx, cy, cz, sh, od, ov,
               sx, sy, sz, sem):
    n_edges = row_hbm.shape[0]
    per_worker = n_edges // N_WORKERS
    n_chunks = per_worker // CHUNK
    sid = lax.axis_index("s")
    wid = sid * N_CORES + lax.axis_index("c")
    wbase = wid * per_worker

    @pl.when(sid == 0)
    def _stage():
        pltpu.sync_copy(px, sx)
        pltpu.sync_copy(py, sy)
        pltpu.sync_copy(pz, sz)

    plsc.subcore_barrier()

    def chunk_body(k, carry):
        base = wbase + k * CHUNK
        pltpu.sync_copy(row_hbm.at[pl.ds(base, CHUNK)], idx_r)
        pltpu.sync_copy(col_hbm.at[pl.ds(base, CHUNK)], idx_c)
        pltpu.sync_copy(shift_hbm.at[pl.ds(base * 3, CHUNK * 3)], sh)
        cps = [pltpu.async_copy(sx.at[idx_r], rx, sem),
               pltpu.async_copy(sy.at[idx_r], ry, sem),
               pltpu.async_copy(sz.at[idx_r], rz, sem),
               pltpu.async_copy(sx.at[idx_c], cx, sem),
               pltpu.async_copy(sy.at[idx_c], cy, sem),
               pltpu.async_copy(sz.at[idx_c], cz, sem)]
        for cp in cps:
            cp.wait()

        lane3 = lax.iota(jnp.int32, LANES) * 3

        def vec_body(j, c2):
            e16 = j * LANES
            s48 = e16 * 3
            m = idx_r[pl.ds(e16, LANES)] != idx_c[pl.ds(e16, LANES)]
            d = []
            for comp, (rv, cv) in enumerate(((rx, cx), (ry, cy), (rz, cz))):
                sid = s48 + comp + lane3
                sv = plsc.load_gather(sh, [sid])
                d.append(rv[pl.ds(e16, LANES)] - cv[pl.ds(e16, LANES)] - sv)
            sq = d[0] * d[0] + d[1] * d[1] + d[2] * d[2]
            bits = plsc.bitcast(sq, jnp.int32)
            y = plsc.bitcast(jnp.int32(0x5F3759DF) - (bits >> 1), jnp.float32)
            for _ in range(3):
                y = y * (1.5 - 0.5 * sq * y * y)
            od[pl.ds(e16, LANES)] = jnp.where(m, sq * y, 0.0)
            ym = jnp.where(m, y, 1.0)
            for comp in range(3):
                plsc.store_scatter(ov, [s48 + comp + lane3], d[comp] * ym)
            return c2

        lax.fori_loop(0, CHUNK // LANES, vec_body, 0)
        pltpu.sync_copy(od, dist_hbm.at[pl.ds(base, CHUNK)])
        pltpu.sync_copy(ov, vec_hbm.at[pl.ds(base * 3, CHUNK * 3)])
        return carry

    lax.fori_loop(0, n_chunks, chunk_body, 0)


def kernel(positions, edge_indices, shift):
    row = edge_indices[0].astype(jnp.int32)
    col = edge_indices[1].astype(jnp.int32)
    px = jnp.asarray(positions[:, 0])
    py = jnp.asarray(positions[:, 1])
    pz = jnp.asarray(positions[:, 2])
    n_edges = row.shape[0]
    mesh = plsc.VectorSubcoreMesh(core_axis_name="c", subcore_axis_name="s")
    fn = pl.kernel(
        _edge_body,
        out_type=(jax.ShapeDtypeStruct((n_edges,), jnp.float32),
                  jax.ShapeDtypeStruct((n_edges * 3,), jnp.float32)),
        mesh=mesh,
        compiler_params=pltpu.CompilerParams(needs_layout_passes=False),
        scratch_types=[
            pltpu.VMEM((CHUNK,), jnp.int32),
            pltpu.VMEM((CHUNK,), jnp.int32),
            pltpu.VMEM((CHUNK,), jnp.float32),
            pltpu.VMEM((CHUNK,), jnp.float32),
            pltpu.VMEM((CHUNK,), jnp.float32),
            pltpu.VMEM((CHUNK,), jnp.float32),
            pltpu.VMEM((CHUNK,), jnp.float32),
            pltpu.VMEM((CHUNK,), jnp.float32),
            pltpu.VMEM((CHUNK * 3,), jnp.float32),
            pltpu.VMEM((CHUNK,), jnp.float32),
            pltpu.VMEM((CHUNK * 3,), jnp.float32),
            pltpu.VMEM_SHARED((50000,), jnp.float32),
            pltpu.VMEM_SHARED((50000,), jnp.float32),
            pltpu.VMEM_SHARED((50000,), jnp.float32),
            pltpu.SemaphoreType.DMA,
        ],
    )
    dist, vec = fn(px, py, pz, row, col, shift.reshape(-1))
    return (edge_indices, dist, vec.reshape(n_edges, 3))


# speedup vs baseline: 19.0573x; 13.6881x over previous
import jax
import jax.numpy as jnp
from jax import lax
from jax.experimental import pallas as pl
from jax.experimental.pallas import tpu as pltpu
from jax.experimental.pallas import tpu_sc as plsc

N_CORES = 2
N_SUBCORES = 16
LANES = 16
N_WORKERS = N_CORES * N_SUBCORES

CHUNK = 2000
N_NODES = 50000


def _edge_body(px, py, pz, row_hbm, col_hbm, shx_hbm, shy_hbm, shz_hbm,
               dist_hbm, vx_hbm, vy_hbm, vz_hbm,
               idx_r, idx_c, rx, ry, rz, cx, cy, cz, shx, shy, shz,
               od, ovx, ovy, ovz, sx, sy, sz, sem):
    n_edges = row_hbm.shape[0]
    per_worker = n_edges // N_WORKERS
    n_chunks = per_worker // CHUNK
    sid = lax.axis_index("s")
    wid = sid * N_CORES + lax.axis_index("c")
    wbase = wid * per_worker

    @pl.when(sid == 0)
    def _stage():
        pltpu.sync_copy(px, sx)
        pltpu.sync_copy(py, sy)
        pltpu.sync_copy(pz, sz)

    plsc.subcore_barrier()

    def chunk_body(k, carry):
        base = wbase + k * CHUNK
        sl = pl.ds(base, CHUNK)
        pltpu.sync_copy(row_hbm.at[sl], idx_r)
        pltpu.sync_copy(col_hbm.at[sl], idx_c)
        pltpu.sync_copy(shx_hbm.at[sl], shx)
        pltpu.sync_copy(shy_hbm.at[sl], shy)
        pltpu.sync_copy(shz_hbm.at[sl], shz)
        cps = [pltpu.async_copy(sx.at[idx_r], rx, sem),
               pltpu.async_copy(sy.at[idx_r], ry, sem),
               pltpu.async_copy(sz.at[idx_r], rz, sem),
               pltpu.async_copy(sx.at[idx_c], cx, sem),
               pltpu.async_copy(sy.at[idx_c], cy, sem),
               pltpu.async_copy(sz.at[idx_c], cz, sem)]
        for cp in cps:
            cp.wait()

        def vec_body(j, c2):
            v = pl.ds(j * LANES, LANES)
            m = idx_r[v] != idx_c[v]
            d0 = rx[v] - cx[v] - shx[v]
            d1 = ry[v] - cy[v] - shy[v]
            d2 = rz[v] - cz[v] - shz[v]
            sq = d0 * d0 + d1 * d1 + d2 * d2
            bits = plsc.bitcast(sq, jnp.int32)
            y = plsc.bitcast(jnp.int32(0x5F3759DF) - (bits >> 1), jnp.float32)
            for _ in range(3):
                y = y * (1.5 - 0.5 * sq * y * y)
            od[v] = jnp.where(m, sq * y, 0.0)
            ym = jnp.where(m, y, 1.0)
            ovx[v] = d0 * ym
            ovy[v] = d1 * ym
            ovz[v] = d2 * ym
            return c2

        lax.fori_loop(0, CHUNK // LANES, vec_body, 0)
        pltpu.sync_copy(od, dist_hbm.at[sl])
        pltpu.sync_copy(ovx, vx_hbm.at[sl])
        pltpu.sync_copy(ovy, vy_hbm.at[sl])
        pltpu.sync_copy(ovz, vz_hbm.at[sl])
        return carry

    lax.fori_loop(0, n_chunks, chunk_body, 0)


def kernel(positions, edge_indices, shift):
    px = jnp.asarray(positions[:, 0])
    py = jnp.asarray(positions[:, 1])
    pz = jnp.asarray(positions[:, 2])
    row = edge_indices[0]
    col = edge_indices[1]
    shx = shift[:, 0]
    shy = shift[:, 1]
    shz = shift[:, 2]
    n_edges = row.shape[0]
    mesh = plsc.VectorSubcoreMesh(core_axis_name="c", subcore_axis_name="s")
    vmem_f = pltpu.VMEM((CHUNK,), jnp.float32)
    fn = pl.kernel(
        _edge_body,
        out_type=(jax.ShapeDtypeStruct((n_edges,), jnp.float32),) * 4,
        mesh=mesh,
        compiler_params=pltpu.CompilerParams(needs_layout_passes=False),
        scratch_types=[
            pltpu.VMEM((CHUNK,), jnp.int32),
            pltpu.VMEM((CHUNK,), jnp.int32),
            vmem_f, vmem_f, vmem_f, vmem_f, vmem_f, vmem_f,
            vmem_f, vmem_f, vmem_f,
            vmem_f, vmem_f, vmem_f, vmem_f,
            pltpu.VMEM_SHARED((N_NODES,), jnp.float32),
            pltpu.VMEM_SHARED((N_NODES,), jnp.float32),
            pltpu.VMEM_SHARED((N_NODES,), jnp.float32),
            pltpu.SemaphoreType.DMA,
        ],
    )
    dist, vx, vy, vz = fn(px, py, pz, row, col, shx, shy, shz)
    vec = jnp.stack([vx, vy, vz], axis=1)
    return (edge_indices, dist, vec)


# speedup vs baseline: 24.5826x; 1.2899x over previous
import jax
import jax.numpy as jnp
from jax import lax
from jax.experimental import pallas as pl
from jax.experimental.pallas import tpu as pltpu
from jax.experimental.pallas import tpu_sc as plsc

N_CORES = 2
N_SUBCORES = 16
LANES = 16
N_WORKERS = N_CORES * N_SUBCORES

CHUNK = 2000
N_NODES = 50000


def _edge_body(px, py, pz, row_hbm, col_hbm, shx_hbm, shy_hbm, shz_hbm,
               dist_hbm, vx_hbm, vy_hbm, vz_hbm,
               idxb0, idxb1, gx0, gx1, gy0, gy1, gz0, gz1,
               shx0, shx1, shy0, shy1, shz0, shz1,
               od0, od1, ovx0, ovx1, ovy0, ovy1, ovz0, ovz1,
               sx, sy, sz, semL, semG0, semG1, semO0, semO1):
    n_edges = row_hbm.shape[0]
    per_worker = n_edges // N_WORKERS
    n_chunks = per_worker // CHUNK
    sid = lax.axis_index("s")
    wid = sid * N_CORES + lax.axis_index("c")
    wbase = wid * per_worker

    idxb = (idxb0, idxb1)
    gx, gy, gz = (gx0, gx1), (gy0, gy1), (gz0, gz1)
    shx, shy, shz = (shx0, shx1), (shy0, shy1), (shz0, shz1)
    od, ovx, ovy, ovz = (od0, od1), (ovx0, ovx1), (ovy0, ovy1), (ovz0, ovz1)
    semG = (semG0, semG1)
    semO = (semO0, semO1)

    @pl.when(sid == 0)
    def _stage():
        pltpu.sync_copy(px, sx)
        pltpu.sync_copy(py, sy)
        pltpu.sync_copy(pz, sz)

    plsc.subcore_barrier()

    def issue_linear(k):
        s = k % 2
        sl = pl.ds(wbase + k * CHUNK, CHUNK)
        return [
            pltpu.async_copy(row_hbm.at[sl], idxb[s].at[pl.ds(0, CHUNK)], semL),
            pltpu.async_copy(col_hbm.at[sl], idxb[s].at[pl.ds(CHUNK, CHUNK)], semL),
            pltpu.async_copy(shx_hbm.at[sl], shx[s], semL),
            pltpu.async_copy(shy_hbm.at[sl], shy[s], semL),
            pltpu.async_copy(shz_hbm.at[sl], shz[s], semL),
        ]

    def issue_gathers(k):
        s = k % 2
        return [
            pltpu.async_copy(sx.at[idxb[s]], gx[s], semG[s]),
            pltpu.async_copy(sy.at[idxb[s]], gy[s], semG[s]),
            pltpu.async_copy(sz.at[idxb[s]], gz[s], semG[s]),
        ]

    def issue_out(k):
        s = k % 2
        sl = pl.ds(wbase + k * CHUNK, CHUNK)
        return [
            pltpu.async_copy(od[s], dist_hbm.at[sl], semO[s]),
            pltpu.async_copy(ovx[s], vx_hbm.at[sl], semO[s]),
            pltpu.async_copy(ovy[s], vy_hbm.at[sl], semO[s]),
            pltpu.async_copy(ovz[s], vz_hbm.at[sl], semO[s]),
        ]

    def compute(k):
        s = k % 2
        cgx, cgy, cgz = gx[s], gy[s], gz[s]
        csx, csy, csz = shx[s], shy[s], shz[s]
        cod, cvx, cvy, cvz = od[s], ovx[s], ovy[s], ovz[s]
        cidx = idxb[s]

        @plsc.parallel_loop(0, CHUNK // LANES, 1, unroll=4)
        def vec_body(j):
            e16 = j * LANES
            v = pl.ds(e16, LANES)
            vc = pl.ds(CHUNK + e16, LANES)
            m = cidx[v] != cidx[vc]
            d0 = cgx[v] - cgx[vc] - csx[v]
            d1 = cgy[v] - cgy[vc] - csy[v]
            d2 = cgz[v] - cgz[vc] - csz[v]
            sq = d0 * d0 + d1 * d1 + d2 * d2
            bits = plsc.bitcast(sq, jnp.int32)
            y = plsc.bitcast(jnp.int32(0x5F3759DF) - (bits >> 1), jnp.float32)
            for _ in range(3):
                y = y * (1.5 - 0.5 * sq * y * y)
            cod[v] = jnp.where(m, sq * y, 0.0)
            ym = jnp.where(m, y, 1.0)
            cvx[v] = d0 * ym
            cvy[v] = d1 * ym
            cvz[v] = d2 * ym

    hL, hG, hO = {}, {}, {}
    hL[0] = issue_linear(0)
    for h in hL[0]:
        h.wait()
    hG[0] = issue_gathers(0)
    if n_chunks > 1:
        hL[1] = issue_linear(1)
    for k in range(n_chunks):
        if k + 1 < n_chunks:
            for h in hL[k + 1]:
                h.wait()
            hG[k + 1] = issue_gathers(k + 1)
        for h in hG[k]:
            h.wait()
        if k + 2 < n_chunks:
            hL[k + 2] = issue_linear(k + 2)
        if k >= 2:
            for h in hO[k - 2]:
                h.wait()
        compute(k)
        hO[k] = issue_out(k)
    for k in (n_chunks - 2, n_chunks - 1):
        for h in hO[k]:
            h.wait()


def kernel(positions, edge_indices, shift):
    px = jnp.asarray(positions[:, 0])
    py = jnp.asarray(positions[:, 1])
    pz = jnp.asarray(positions[:, 2])
    row = edge_indices[0]
    col = edge_indices[1]
    shx = shift[:, 0]
    shy = shift[:, 1]
    shz = shift[:, 2]
    n_edges = row.shape[0]
    mesh = plsc.VectorSubcoreMesh(core_axis_name="c", subcore_axis_name="s")
    f = pltpu.VMEM((CHUNK,), jnp.float32)
    f2 = pltpu.VMEM((2 * CHUNK,), jnp.float32)
    i2 = pltpu.VMEM((2 * CHUNK,), jnp.int32)
    fn = pl.kernel(
        _edge_body,
        out_type=(jax.ShapeDtypeStruct((n_edges,), jnp.float32),) * 4,
        mesh=mesh,
        compiler_params=pltpu.CompilerParams(needs_layout_passes=False),
        scratch_types=[
            i2, i2,
            f2, f2, f2, f2, f2, f2,
            f, f, f, f, f, f,
            f, f, f, f, f, f, f, f,
            pltpu.VMEM_SHARED((N_NODES,), jnp.float32),
            pltpu.VMEM_SHARED((N_NODES,), jnp.float32),
            pltpu.VMEM_SHARED((N_NODES,), jnp.float32),
            pltpu.SemaphoreType.DMA,
            pltpu.SemaphoreType.DMA,
            pltpu.SemaphoreType.DMA,
            pltpu.SemaphoreType.DMA,
            pltpu.SemaphoreType.DMA,
        ],
    )
    dist, vx, vy, vz = fn(px, py, pz, row, col, shx, shy, shz)
    vec = jnp.stack([vx, vy, vz], axis=1)
    return (edge_indices, dist, vec)
